# baseline (speedup 1.0000x reference)
import functools

import jax
import jax.numpy as jnp
from jax import lax
from jax.experimental import pallas as pl
from jax.experimental.pallas import tpu as pltpu
from jax.experimental.pallas import tpu_sc as plsc

_EPS = 1e-8


def _sqrt16(x):
    xc = jnp.maximum(x, jnp.float32(1e-30))
    i = plsc.bitcast(xc, jnp.int32)
    i = jnp.int32(0x5F3759DF) - (i >> 1)
    y = plsc.bitcast(i, jnp.float32)
    half = jnp.float32(0.5) * xc
    for _ in range(3):
        y = y * (jnp.float32(1.5) - half * y * y)
    return xc * y


@functools.partial(jax.jit, static_argnames=())
def kernel(user_ids, item_ids, user_table, item_table):
    B = user_ids.shape[0]
    D = user_table.shape[1]
    NC, NS, L = 2, 16, 16
    NW = NC * NS
    assert B % (NW * L) == 0
    b_per_w = B // NW
    groups = b_per_w // L

    mesh = plsc.VectorSubcoreMesh(
        core_axis_name="c", subcore_axis_name="s",
        num_cores=NC, num_subcores=NS)

    @functools.partial(
        pl.kernel,
        out_type=jax.ShapeDtypeStruct((B,), jnp.float32),
        mesh=mesh,
        compiler_params=pltpu.CompilerParams(
            needs_layout_passes=False, use_tc_tiling_on_sc=False),
        scratch_types=[
            pltpu.VMEM((b_per_w,), jnp.int32),
            pltpu.VMEM((b_per_w,), jnp.int32),
            pltpu.VMEM((b_per_w, D), jnp.float32),
            pltpu.VMEM((b_per_w, D), jnp.float32),
            pltpu.VMEM((b_per_w,), jnp.float32),
            pltpu.SemaphoreType.DMA,
        ],
    )
    def _cosine_sc(uid_hbm, iid_hbm, ut_hbm, it_hbm, out_hbm,
                   uid_v, iid_v, urows, irows, out_v, sem):
        wid = lax.axis_index("s") * NC + lax.axis_index("c")
        base = wid * b_per_w
        pltpu.sync_copy(uid_hbm.at[pl.ds(base, b_per_w)], uid_v)
        pltpu.sync_copy(iid_hbm.at[pl.ds(base, b_per_w)], iid_v)
        cu = pltpu.async_copy(ut_hbm.at[uid_v], urows, sem)
        ci = pltpu.async_copy(it_hbm.at[iid_v], irows, sem)
        cu.wait()
        ci.wait()

        lane = lax.iota(jnp.int32, L)

        def group_body(g, _):
            row = g * L + lane
            zero = jnp.zeros((L,), jnp.float32)
            dot, uu, ii = zero, zero, zero
            for d in range(D):
                col = jnp.full((L,), d, jnp.int32)
                u = plsc.load_gather(urows, [row, col])
                v = plsc.load_gather(irows, [row, col])
                dot = dot + u * v
                uu = uu + u * u
                ii = ii + v * v
            n1 = jnp.maximum(_sqrt16(uu), jnp.float32(_EPS))
            n2 = jnp.maximum(_sqrt16(ii), jnp.float32(_EPS))
            out_v[pl.ds(g * L, L)] = dot / (n1 * n2)
            return 0

        lax.fori_loop(0, groups, group_body, 0)
        pltpu.sync_copy(out_v, out_hbm.at[pl.ds(base, b_per_w)])

    return _cosine_sc(user_ids, item_ids, user_table, item_table)
